# baseline (speedup 1.0000x reference)
import functools

import jax
import jax.numpy as jnp
from jax import lax
from jax.experimental import pallas as pl
from jax.experimental.pallas import tpu as pltpu
from jax.experimental.pallas import tpu_sc as plsc

FRAME = 3 * 112 * 112
S_SPLIT = 6
FINE = FRAME // S_SPLIT
N_OUT_FRAMES = 8 * 16
N_OUT_ROWS = N_OUT_FRAMES * S_SPLIT
N_TILES = 32
ROWS_PER_TILE = N_OUT_ROWS // N_TILES
CHUNK = 8
N_CHUNKS = ROWS_PER_TILE // CHUNK


def _sc_gather(x2, fidx):
  mesh = plsc.VectorSubcoreMesh(core_axis_name="c", subcore_axis_name="s")

  @functools.partial(
      pl.kernel,
      mesh=mesh,
      out_type=jax.ShapeDtypeStruct((N_OUT_ROWS, FINE), jnp.float32),
      scratch_types=[
          pltpu.VMEM((ROWS_PER_TILE,), jnp.int32),
          pltpu.VMEM((CHUNK, FINE), jnp.float32),
          pltpu.VMEM((CHUNK, FINE), jnp.float32),
          pltpu.SemaphoreType.DMA,
          pltpu.SemaphoreType.DMA,
      ],
  )
  def k(x_hbm, fidx_hbm, out_hbm, idx_v, buf_a, buf_b, gsem, ssem):
    wid = lax.axis_index("s") * 2 + lax.axis_index("c")
    base = wid * ROWS_PER_TILE
    pltpu.sync_copy(fidx_hbm.at[pl.ds(base, ROWS_PER_TILE)], idx_v)
    bufs = (buf_a, buf_b)
    gathers = [None] * N_CHUNKS
    scatters = [None, None]
    for c in range(N_CHUNKS):
      buf = bufs[c % 2]
      if scatters[c % 2] is not None:
        scatters[c % 2].wait()
        scatters[c % 2] = None
      gathers[c] = pltpu.async_copy(
          x_hbm.at[idx_v.at[pl.ds(c * CHUNK, CHUNK)]], buf, gsem)
      if c == 0:
        continue
      gathers[c - 1].wait()
      scatters[(c - 1) % 2] = pltpu.async_copy(
          bufs[(c - 1) % 2],
          out_hbm.at[pl.ds(base + (c - 1) * CHUNK, CHUNK)],
          ssem)
    last = N_CHUNKS - 1
    gathers[last].wait()
    pltpu.async_copy(
        bufs[last % 2],
        out_hbm.at[pl.ds(base + last * CHUNK, CHUNK)],
        ssem).wait()
    if scatters[(last - 1) % 2] is not None:
      scatters[(last - 1) % 2].wait()

  return k(x2, fidx)


def kernel(x):
  B, T = x.shape[0], x.shape[1]
  fnum = 16
  start, end = 0, T - 1
  fid_base = jnp.linspace(start, end, fnum).astype(jnp.int32)
  step = int((end - start) / fnum)
  if step != 0:
    key = jax.random.key(42)
    y = jax.random.randint(key, (B, fnum), 0, step, dtype=jnp.int32)
    y = y.at[:, fnum - 1].set(0)
  else:
    y = jnp.zeros((B, fnum), dtype=jnp.int32)
  fid = fid_base[None, :] + y
  frame_global = (jnp.arange(B, dtype=jnp.int32)[:, None] * T
                  + fid).reshape(-1)
  fine = (frame_global[:, None] * S_SPLIT
          + jnp.arange(S_SPLIT, dtype=jnp.int32)[None, :]).reshape(-1)
  x2 = x.reshape(B * T * S_SPLIT, FINE)
  out2 = _sc_gather(x2, fine)
  return out2.reshape(B, fnum, *x.shape[2:])
